# baseline (speedup 1.0000x reference)
import jax
import jax.numpy as jnp
from jax import lax
from jax.experimental import pallas as pl
from jax.experimental.pallas import tpu as pltpu
from jax.experimental.pallas import tpu_sc as plsc

N = 8192
C = 128
NSAMP = 16
NHEAD = 8
DH = 16
DFF = 512
BQ = 1024


def _ln(x, g, b):
    m = jnp.mean(x, axis=-1, keepdims=True)
    var = jnp.mean((x - m) ** 2, axis=-1, keepdims=True)
    return (x - m) / jnp.sqrt(var + 1e-5) * g + b


SC_NC = 2
SC_NS = 16
SC_NW = SC_NC * SC_NS
GCH = 128


def _sc_gather_body(table_hbm, idx_hbm, out_hbm,
                    idx_all, rows0, rows1, rows2, rows3,
                    sem0, sem1, sem2, sem3, tsem0, tsem1, tsem2, tsem3):
    wid = lax.axis_index("s") * SC_NC + lax.axis_index("c")
    bpw = idx_hbm.shape[0] // SC_NW
    base = wid * bpw
    pltpu.sync_copy(idx_hbm.at[pl.ds(base, bpw)], idx_all)
    bufs = ((rows0, sem0, tsem0), (rows1, sem1, tsem1),
            (rows2, sem2, tsem2), (rows3, sem3, tsem3))

    def step(i, carry):
        off = i * (4 * GCH)
        cps = [pltpu.async_copy(
                   table_hbm.at[idx_all.at[pl.ds(off + j * GCH, GCH)]],
                   rows, sem)
               for j, (rows, sem, _) in enumerate(bufs)]
        sts = []
        for j, (rows, _, tsem) in enumerate(bufs):
            cps[j].wait()
            sts.append(pltpu.async_copy(
                rows, out_hbm.at[pl.ds(base + off + j * GCH, GCH)], tsem))
        for st in sts:
            st.wait()
        return carry

    lax.fori_loop(0, bpw // (4 * GCH), step, 0)


def _sc_gather(table, idx_flat):
    b = idx_flat.shape[0]
    nc = table.shape[1]
    bpw = b // SC_NW
    kfn = pl.kernel(
        _sc_gather_body,
        out_type=jax.ShapeDtypeStruct((b, nc), table.dtype),
        mesh=plsc.VectorSubcoreMesh(core_axis_name="c", subcore_axis_name="s"),
        scratch_types=[
            pltpu.VMEM((bpw,), jnp.int32),
            pltpu.VMEM((GCH, nc), table.dtype),
            pltpu.VMEM((GCH, nc), table.dtype),
            pltpu.VMEM((GCH, nc), table.dtype),
            pltpu.VMEM((GCH, nc), table.dtype),
            pltpu.SemaphoreType.DMA,
            pltpu.SemaphoreType.DMA,
            pltpu.SemaphoreType.DMA,
            pltpu.SemaphoreType.DMA,
            pltpu.SemaphoreType.DMA,
            pltpu.SemaphoreType.DMA,
            pltpu.SemaphoreType.DMA,
            pltpu.SemaphoreType.DMA,
        ],
    )
    return kfn(table, idx_flat)


BK = 512


_I32MAX = 0x7FFFFFFF


def _extract16(key):
    cols = []
    m = None
    for _ in range(NSAMP):
        m = jnp.min(key, axis=1, keepdims=True)
        cols.append(m & 0x1FFF)
        key = jnp.where(key == m, _I32MAX, key)
    return jnp.concatenate(cols, axis=1), m


def _knn_body(xq_ref, xkT_ref, idx_ref):
    xq = xq_ref[...]
    xkT = xkT_ref[...]
    sqq = jnp.sum(xq * xq, axis=1, keepdims=True)
    sqk = jnp.sum(xkT * xkT, axis=0, keepdims=True)
    mm = jnp.dot(xq, xkT, preferred_element_type=jnp.float32)
    d2 = jnp.maximum(sqq + sqk - 2.0 * mm, 0.0)
    bits = jax.lax.bitcast_convert_type(d2, jnp.int32)
    lane = jax.lax.broadcasted_iota(jnp.int32, (BK, N), 1)
    key = (bits & jnp.int32(~0x1FFF)) | lane
    a = b = c = d = e = jnp.full((BK, 128), _I32MAX, jnp.int32)
    for ch in range(N // 128):
        x = key[:, ch * 128:(ch + 1) * 128]
        e = jnp.minimum(e, jnp.maximum(d, x))
        d = jnp.minimum(d, jnp.maximum(c, x))
        c = jnp.minimum(c, jnp.maximum(b, x))
        b = jnp.minimum(b, jnp.maximum(a, x))
        a = jnp.minimum(a, x)
    laneio = jax.lax.broadcasted_iota(jnp.int32, (BK, 128), 1)
    cols = []
    tau = None
    for _ in range(NSAMP):
        tau = jnp.min(a, axis=1, keepdims=True)
        cols.append(tau & 0x1FFF)
        msk = laneio == (tau & 0x7F)
        a = jnp.where(msk, b, a)
        b = jnp.where(msk, c, b)
        c = jnp.where(msk, d, c)
        d = jnp.where(msk, _I32MAX, d)
    idx16 = jnp.concatenate(cols, axis=1)
    need_full = jnp.any(e <= tau)
    idx_ref[...] = lax.cond(need_full,
                            lambda: _extract16(key)[0],
                            lambda: idx16)


def _knn(xq_pad8, xyzT_pad8):
    nq = xq_pad8.shape[0]
    return pl.pallas_call(
        _knn_body,
        grid=(nq // BK,),
        in_specs=[
            pl.BlockSpec((BK, 8), lambda i: (i, 0)),
            pl.BlockSpec((8, N), lambda i: (0, 0)),
        ],
        out_specs=pl.BlockSpec((BK, NSAMP), lambda i: (i, 0)),
        out_shape=jax.ShapeDtypeStruct((nq, NSAMP), jnp.int32),
    )(xq_pad8, xyzT_pad8)


def _xyzw_body(xyz_ref, w_ref, o_ref):
    o_ref[...] = jnp.dot(xyz_ref[...], w_ref[...],
                         preferred_element_type=jnp.float32)


def _xyzw(xyz_pad, w_pad):
    return pl.pallas_call(
        _xyzw_body,
        out_shape=jax.ShapeDtypeStruct((N, C), jnp.float32),
    )(xyz_pad, w_pad)


def _layer_body(out_ref, mem_ref, g_ref, xq_ref, qpe_ref,
                peW2_ref, peb1_ref, peb2_ref,
                wq_ref, bq_ref, wk_ref, bk_ref, wv_ref, bv_ref,
                wo_ref, bo_ref, gca_ref, bca_ref,
                w1_ref, b1_ref, w2_ref, b2_ref,
                gffn_ref, bffn_ref, gln_ref, bln_ref,
                onew_ref):
    out = out_ref[...]
    mem = mem_ref[...]
    g = g_ref[...]
    xq = xq_ref[...]

    xq_rep = jnp.broadcast_to(
        xq[:, None, :], (BQ, NSAMP, C)).reshape(BQ * NSAMP, C)
    h = jnp.maximum(g - xq_rep + peb1_ref[...], 0.0)
    knn_pe = jnp.dot(h, peW2_ref[...],
                     preferred_element_type=jnp.float32) + peb2_ref[...]

    t2 = _ln(out, gca_ref[...], bca_ref[...])
    q = jnp.dot(t2 + qpe_ref[...], wq_ref[...],
                preferred_element_type=jnp.float32) + bq_ref[...]
    k = jnp.dot(mem + knn_pe, wk_ref[...],
                preferred_element_type=jnp.float32) + bk_ref[...]
    v = jnp.dot(mem, wv_ref[...],
                preferred_element_type=jnp.float32) + bv_ref[...]

    qb = jnp.broadcast_to(
        q[:, None, :], (BQ, NSAMP, C)).reshape(BQ * NSAMP, C)
    prod = qb * k
    seg = (jax.lax.broadcasted_iota(jnp.int32, (C, NHEAD), 0) // DH
           == jax.lax.broadcasted_iota(jnp.int32, (C, NHEAD), 1))
    logits = jnp.dot(prod, seg.astype(jnp.float32),
                     preferred_element_type=jnp.float32) * (1.0 / 4.0)
    logits = logits.reshape(BQ, NSAMP, NHEAD)
    mx = jnp.max(logits, axis=1, keepdims=True)
    e = jnp.exp(logits - mx)
    att = e / jnp.sum(e, axis=1, keepdims=True)
    attb = jnp.dot(att.reshape(BQ * NSAMP, NHEAD),
                   seg.astype(jnp.float32).T,
                   preferred_element_type=jnp.float32)
    o = jnp.sum((attb * v).reshape(BQ, NSAMP, C), axis=1)

    out = out + jnp.dot(o, wo_ref[...],
                        preferred_element_type=jnp.float32) + bo_ref[...]
    t = _ln(out, gffn_ref[...], bffn_ref[...])
    t = jnp.maximum(jnp.dot(t, w1_ref[...],
                            preferred_element_type=jnp.float32)
                    + b1_ref[...], 0.0)
    t = jnp.dot(t, w2_ref[...],
                preferred_element_type=jnp.float32) + b2_ref[...]
    out = out + t
    onew_ref[...] = _ln(out, gln_ref[...], bln_ref[...])


def _layer(out, mem, g, xyzw, qpe, peW2, peb1, peb2,
           wq, bq_, wk, bk_, wv, bv_, wo, bo_, gca, bca,
           w1, b1_, w2, b2_, gffn, bffn, gln, bln):
    nr = out.shape[0]
    nb = nr // BQ
    row = lambda i: (i, 0)
    full = lambda i: (0, 0)
    bs_row = pl.BlockSpec((BQ, C), row)
    bs_mem = pl.BlockSpec((BQ * NSAMP, C), row)

    def fullspec(shape):
        return pl.BlockSpec(shape, full)

    return pl.pallas_call(
        _layer_body,
        grid=(nb,),
        in_specs=[
            bs_row,
            bs_mem,
            bs_mem,
            bs_row,
            fullspec((1, C)),
            fullspec((C, C)),
            fullspec((1, C)),
            fullspec((1, C)),
            fullspec((C, C)), fullspec((1, C)),
            fullspec((C, C)), fullspec((1, C)),
            fullspec((C, C)), fullspec((1, C)),
            fullspec((C, C)), fullspec((1, C)),
            fullspec((1, C)), fullspec((1, C)),
            fullspec((C, DFF)), fullspec((1, DFF)),
            fullspec((DFF, C)), fullspec((1, C)),
            fullspec((1, C)), fullspec((1, C)),
            fullspec((1, C)), fullspec((1, C)),
        ],
        out_specs=bs_row,
        out_shape=jax.ShapeDtypeStruct((nr, C), jnp.float32),
    )(out, mem, g, xyzw, qpe, peW2, peb1, peb2,
      wq, bq_, wk, bk_, wv, bv_, wo, bo_, gca, bca,
      w1, b1_, w2, b2_, gffn, bffn, gln, bln)


def kernel(feature_0, xyz_0, bs, v, pe_W1, pe_b1, pe_W2, pe_b2,
           Wq, bq, Wk, bk, Wv, bv, Wo, bo, g_ca, b_ca,
           W1, b1, W2, b2, g_ffn, b_ffn, g_ln, b_ln):
    f = feature_0.shape[1]
    one = (jnp.asarray(bs * v, feature_0.dtype)
           / jnp.asarray(feature_0.shape[0], feature_0.dtype))
    feat = (feature_0 * one).transpose(0, 2, 3, 1).reshape(N, C)
    xyz = xyz_0.reshape(N, 3)

    xyz_pad = jnp.pad(xyz, ((0, 0), (0, 5)))
    idx = _knn(xyz_pad, xyz_pad.T)

    w_pad = jnp.pad(pe_W1, ((0, 5), (0, 0)))
    xyzw = _xyzw(xyz_pad, w_pad)

    flat_idx = idx.reshape(N * NSAMP)
    g = _sc_gather(xyzw, flat_idx)

    qpe = (jnp.maximum(pe_b1, 0.0) @ pe_W2 + pe_b2).reshape(1, C)

    r2 = lambda x: x.reshape(1, -1)
    out = feat
    for i in range(Wq.shape[0]):
        mem = _sc_gather(out, flat_idx)
        out = _layer(out, mem, g, xyzw, qpe,
                     pe_W2, r2(pe_b1), r2(pe_b2),
                     Wq[i], r2(bq[i]), Wk[i], r2(bk[i]),
                     Wv[i], r2(bv[i]), Wo[i], r2(bo[i]),
                     r2(g_ca[i]), r2(b_ca[i]),
                     W1[i], r2(b1[i]), W2[i], r2(b2[i]),
                     r2(g_ffn[i]), r2(b_ffn[i]),
                     r2(g_ln[i]), r2(b_ln[i]))

    return out.reshape(1, 8, 32, 32, C).transpose(0, 1, 4, 2, 3).reshape(8, C, 32, 32)


# speedup vs baseline: 1.0008x; 1.0008x over previous
import jax
import jax.numpy as jnp
from jax import lax
from jax.experimental import pallas as pl
from jax.experimental.pallas import tpu as pltpu
from jax.experimental.pallas import tpu_sc as plsc

N = 8192
C = 128
NSAMP = 16
NHEAD = 8
DH = 16
DFF = 512
BQ = 1024


def _ln(x, g, b):
    m = jnp.mean(x, axis=-1, keepdims=True)
    var = jnp.mean((x - m) ** 2, axis=-1, keepdims=True)
    return (x - m) / jnp.sqrt(var + 1e-5) * g + b


SC_NC = 2
SC_NS = 16
SC_NW = SC_NC * SC_NS
GCH = 128


def _sc_gather_body(table_hbm, idx_hbm, out_hbm,
                    idx_all, rows0, rows1, rows2, rows3,
                    sem0, sem1, sem2, sem3, tsem0, tsem1, tsem2, tsem3):
    wid = lax.axis_index("s") * SC_NC + lax.axis_index("c")
    bpw = idx_hbm.shape[0] // SC_NW
    base = wid * bpw
    pltpu.sync_copy(idx_hbm.at[pl.ds(base, bpw)], idx_all)
    bufs = ((rows0, sem0, tsem0), (rows1, sem1, tsem1),
            (rows2, sem2, tsem2), (rows3, sem3, tsem3))

    def step(i, carry):
        off = i * (4 * GCH)
        cps = [pltpu.async_copy(
                   table_hbm.at[idx_all.at[pl.ds(off + j * GCH, GCH)]],
                   rows, sem)
               for j, (rows, sem, _) in enumerate(bufs)]
        sts = []
        for j, (rows, _, tsem) in enumerate(bufs):
            cps[j].wait()
            sts.append(pltpu.async_copy(
                rows, out_hbm.at[pl.ds(base + off + j * GCH, GCH)], tsem))
        for st in sts:
            st.wait()
        return carry

    lax.fori_loop(0, bpw // (4 * GCH), step, 0)


def _sc_gather(table, idx_flat):
    b = idx_flat.shape[0]
    nc = table.shape[1]
    bpw = b // SC_NW
    kfn = pl.kernel(
        _sc_gather_body,
        out_type=jax.ShapeDtypeStruct((b, nc), table.dtype),
        mesh=plsc.VectorSubcoreMesh(core_axis_name="c", subcore_axis_name="s"),
        scratch_types=[
            pltpu.VMEM((bpw,), jnp.int32),
            pltpu.VMEM((GCH, nc), table.dtype),
            pltpu.VMEM((GCH, nc), table.dtype),
            pltpu.VMEM((GCH, nc), table.dtype),
            pltpu.VMEM((GCH, nc), table.dtype),
            pltpu.SemaphoreType.DMA,
            pltpu.SemaphoreType.DMA,
            pltpu.SemaphoreType.DMA,
            pltpu.SemaphoreType.DMA,
            pltpu.SemaphoreType.DMA,
            pltpu.SemaphoreType.DMA,
            pltpu.SemaphoreType.DMA,
            pltpu.SemaphoreType.DMA,
        ],
    )
    return kfn(table, idx_flat)


BK = 512


_I32MAX = 0x7FFFFFFF


def _extract16(key):
    cols = []
    m = None
    for _ in range(NSAMP):
        m = jnp.min(key, axis=1, keepdims=True)
        cols.append(m & 0x1FFF)
        key = jnp.where(key == m, _I32MAX, key)
    return jnp.concatenate(cols, axis=1), m


def _knn_body(xq_ref, xkT_ref, idx_ref):
    xq = xq_ref[...]
    xkT = xkT_ref[...]
    sqq = jnp.sum(xq * xq, axis=1, keepdims=True)
    sqk = jnp.sum(xkT * xkT, axis=0, keepdims=True)
    mm = jnp.dot(xq, xkT, preferred_element_type=jnp.float32)
    d2 = jnp.maximum(sqq + sqk - 2.0 * mm, 0.0)
    bits = jax.lax.bitcast_convert_type(d2, jnp.int32)
    lane = jax.lax.broadcasted_iota(jnp.int32, (BK, N), 1)
    key = (bits & jnp.int32(~0x1FFF)) | lane
    a = b = c = d = e = jnp.full((BK, 128), _I32MAX, jnp.int32)
    for ch in range(N // 128):
        x = key[:, ch * 128:(ch + 1) * 128]
        e = jnp.minimum(e, jnp.maximum(d, x))
        d = jnp.minimum(d, jnp.maximum(c, x))
        c = jnp.minimum(c, jnp.maximum(b, x))
        b = jnp.minimum(b, jnp.maximum(a, x))
        a = jnp.minimum(a, x)
    laneio = jax.lax.broadcasted_iota(jnp.int32, (BK, 128), 1)
    cols = []
    tau = None
    for _ in range(NSAMP):
        tau = jnp.min(a, axis=1, keepdims=True)
        cols.append(tau & 0x1FFF)
        msk = laneio == (tau & 0x7F)
        a = jnp.where(msk, b, a)
        b = jnp.where(msk, c, b)
        c = jnp.where(msk, d, c)
        d = jnp.where(msk, _I32MAX, d)
    idx16 = jnp.concatenate(cols, axis=1)
    need_full = jnp.any(e <= tau)
    idx_ref[...] = lax.cond(need_full,
                            lambda: _extract16(key)[0],
                            lambda: idx16)


def _knn(xq_pad8, xyzT_pad8):
    nq = xq_pad8.shape[0]
    return pl.pallas_call(
        _knn_body,
        grid=(nq // BK,),
        in_specs=[
            pl.BlockSpec((BK, 8), lambda i: (i, 0)),
            pl.BlockSpec((8, N), lambda i: (0, 0)),
        ],
        out_specs=pl.BlockSpec((BK, NSAMP), lambda i: (i, 0)),
        out_shape=jax.ShapeDtypeStruct((nq, NSAMP), jnp.int32),
    )(xq_pad8, xyzT_pad8)


def _xyzw_body(xyz_ref, w_ref, o_ref):
    o_ref[...] = jnp.dot(xyz_ref[...], w_ref[...],
                         preferred_element_type=jnp.float32)


def _xyzw(xyz_pad, w_pad):
    return pl.pallas_call(
        _xyzw_body,
        out_shape=jax.ShapeDtypeStruct((N, C), jnp.float32),
    )(xyz_pad, w_pad)


def _layer_body(out_ref, mem_ref, g_ref, xq_ref, qpe_ref,
                peW2_ref, peb1_ref, peb2_ref,
                wq_ref, bq_ref, wk_ref, bk_ref, wv_ref, bv_ref,
                wo_ref, bo_ref, gca_ref, bca_ref,
                w1_ref, b1_ref, w2_ref, b2_ref,
                gffn_ref, bffn_ref, gln_ref, bln_ref,
                onew_ref):
    out = out_ref[...]
    mem = mem_ref[...]
    g = g_ref[...]
    xq = xq_ref[...]

    xq_rep = jnp.broadcast_to(
        xq[:, None, :], (BQ, NSAMP, C)).reshape(BQ * NSAMP, C)
    h = jnp.maximum(g - xq_rep + peb1_ref[...], 0.0)
    knn_pe = jnp.dot(h, peW2_ref[...],
                     preferred_element_type=jnp.float32) + peb2_ref[...]

    t2 = _ln(out, gca_ref[...], bca_ref[...])
    q = jnp.dot(t2 + qpe_ref[...], wq_ref[...],
                preferred_element_type=jnp.float32) + bq_ref[...]
    k = jnp.dot(mem + knn_pe, wk_ref[...],
                preferred_element_type=jnp.float32) + bk_ref[...]
    v = jnp.dot(mem, wv_ref[...],
                preferred_element_type=jnp.float32) + bv_ref[...]

    qb = jnp.broadcast_to(
        q[:, None, :], (BQ, NSAMP, C)).reshape(BQ * NSAMP, C)
    prod = qb * k
    seg = (jax.lax.broadcasted_iota(jnp.int32, (C, NHEAD), 0) // DH
           == jax.lax.broadcasted_iota(jnp.int32, (C, NHEAD), 1))
    logits = jnp.dot(prod, seg.astype(jnp.float32),
                     preferred_element_type=jnp.float32) * (1.0 / 4.0)
    logits = logits.reshape(BQ, NSAMP, NHEAD)
    mx = jnp.max(logits, axis=1, keepdims=True)
    e = jnp.exp(logits - mx)
    att = e / jnp.sum(e, axis=1, keepdims=True)
    attb = jnp.dot(att.reshape(BQ * NSAMP, NHEAD),
                   seg.astype(jnp.float32).T,
                   preferred_element_type=jnp.float32)
    o = jnp.sum((attb * v).reshape(BQ, NSAMP, C), axis=1)

    out = out + jnp.dot(o, wo_ref[...],
                        preferred_element_type=jnp.float32) + bo_ref[...]
    t = _ln(out, gffn_ref[...], bffn_ref[...])
    t = jnp.maximum(jnp.dot(t, w1_ref[...],
                            preferred_element_type=jnp.float32)
                    + b1_ref[...], 0.0)
    t = jnp.dot(t, w2_ref[...],
                preferred_element_type=jnp.float32) + b2_ref[...]
    out = out + t
    onew_ref[...] = _ln(out, gln_ref[...], bln_ref[...])


def _layer(out, mem, g, xyzw, qpe, peW2, peb1, peb2,
           wq, bq_, wk, bk_, wv, bv_, wo, bo_, gca, bca,
           w1, b1_, w2, b2_, gffn, bffn, gln, bln):
    nr = out.shape[0]
    nb = nr // BQ
    row = lambda i: (i, 0)
    full = lambda i: (0, 0)
    bs_row = pl.BlockSpec((BQ, C), row)
    bs_mem = pl.BlockSpec((BQ * NSAMP, C), row)

    def fullspec(shape):
        return pl.BlockSpec(shape, full)

    return pl.pallas_call(
        _layer_body,
        grid=(nb,),
        in_specs=[
            bs_row,
            bs_mem,
            bs_mem,
            bs_row,
            fullspec((1, C)),
            fullspec((C, C)),
            fullspec((1, C)),
            fullspec((1, C)),
            fullspec((C, C)), fullspec((1, C)),
            fullspec((C, C)), fullspec((1, C)),
            fullspec((C, C)), fullspec((1, C)),
            fullspec((C, C)), fullspec((1, C)),
            fullspec((1, C)), fullspec((1, C)),
            fullspec((C, DFF)), fullspec((1, DFF)),
            fullspec((DFF, C)), fullspec((1, C)),
            fullspec((1, C)), fullspec((1, C)),
            fullspec((1, C)), fullspec((1, C)),
        ],
        out_specs=bs_row,
        out_shape=jax.ShapeDtypeStruct((nr, C), jnp.float32),
    )(out, mem, g, xyzw, qpe, peW2, peb1, peb2,
      wq, bq_, wk, bk_, wv, bv_, wo, bo_, gca, bca,
      w1, b1_, w2, b2_, gffn, bffn, gln, bln)


def kernel(feature_0, xyz_0, bs, v, pe_W1, pe_b1, pe_W2, pe_b2,
           Wq, bq, Wk, bk, Wv, bv, Wo, bo, g_ca, b_ca,
           W1, b1, W2, b2, g_ffn, b_ffn, g_ln, b_ln):
    one = (jnp.asarray(bs * v, feature_0.dtype)
           / jnp.asarray(feature_0.shape[0], feature_0.dtype))
    feat = (feature_0 * one).transpose(0, 2, 3, 1).reshape(N, C)
    xyz = xyz_0.reshape(N, 3)

    xyz_pad = jnp.pad(xyz, ((0, 0), (0, 5)))
    idx = _knn(xyz_pad, xyz_pad.T)

    w_pad = jnp.pad(pe_W1, ((0, 5), (0, 0)))
    xyzw = _xyzw(xyz_pad, w_pad)

    flat_idx = idx.reshape(N * NSAMP)
    g = _sc_gather(xyzw, flat_idx)

    qpe = (jnp.maximum(pe_b1, 0.0) @ pe_W2 + pe_b2).reshape(1, C)

    r2 = lambda x: x.reshape(1, -1)
    out = feat
    for i in range(Wq.shape[0]):
        mem = _sc_gather(out, flat_idx)
        out = _layer(out, mem, g, xyzw, qpe,
                     pe_W2, r2(pe_b1), r2(pe_b2),
                     Wq[i], r2(bq[i]), Wk[i], r2(bk[i]),
                     Wv[i], r2(bv[i]), Wo[i], r2(bo[i]),
                     r2(g_ca[i]), r2(b_ca[i]),
                     W1[i], r2(b1[i]), W2[i], r2(b2[i]),
                     r2(g_ffn[i]), r2(b_ffn[i]),
                     r2(g_ln[i]), r2(b_ln[i]))

    return out.reshape(1, 8, 32, 32, C).transpose(0, 1, 4, 2, 3).reshape(8, C, 32, 32)
